# Initial kernel scaffold; baseline (speedup 1.0000x reference)
import jax
import jax.numpy as jnp
from jax.experimental import pallas as pl


def kernel(x, temporal_step, sequence_pattern, temporal_pe, spatial_pe, sequence_pe):
    raise NotImplementedError("write your pallas kernel here")


# speedup vs baseline: 1.6057x; 1.6057x over previous
import jax
import jax.numpy as jnp
from jax.experimental import pallas as pl
from jax.experimental.pallas import tpu as pltpu


def _body(step_ref, pat_ref, x_ref, tpe_ref, spe_ref, qpe_ref, o_ref):
    b = pl.program_id(1)
    s = step_ref[0]
    idx = pat_ref[b] % 64
    td = tpe_ref.shape[1]
    sd = spe_ref.shape[2]
    t_row = tpe_ref[s, :]
    q_row = qpe_ref[idx, :]
    o_ref[..., :td] = x_ref[..., :td] + t_row[None, None, None, :]
    o_ref[..., td:td + sd] = x_ref[..., td:td + sd] + spe_ref[...][None]
    o_ref[..., td + sd:] = x_ref[..., td + sd:] + q_row[None, None, None, :]


def kernel(x, temporal_step, sequence_pattern, temporal_pe, spatial_pe, sequence_pe):
    B, H, W, D = x.shape
    TD = temporal_pe.shape[1]
    SD = spatial_pe.shape[2]
    QD = sequence_pe.shape[1]
    RB = 16
    R = H // RB

    step = jnp.asarray(temporal_step, jnp.int32).reshape(1)
    pat = jnp.asarray(sequence_pattern, jnp.int32)

    grid_spec = pltpu.PrefetchScalarGridSpec(
        num_scalar_prefetch=2,
        grid=(R, B),
        in_specs=[
            pl.BlockSpec((1, RB, W, D), lambda r, b, *_: (b, r, 0, 0)),
            pl.BlockSpec(temporal_pe.shape, lambda r, b, *_: (0, 0)),
            pl.BlockSpec((RB, W, SD), lambda r, b, *_: (r, 0, 0)),
            pl.BlockSpec(sequence_pe.shape, lambda r, b, *_: (0, 0)),
        ],
        out_specs=pl.BlockSpec((1, RB, W, D), lambda r, b, *_: (b, r, 0, 0)),
    )
    return pl.pallas_call(
        _body,
        grid_spec=grid_spec,
        out_shape=jax.ShapeDtypeStruct(x.shape, x.dtype),
    )(step, pat, x, temporal_pe, spatial_pe, sequence_pe)


# speedup vs baseline: 1.6483x; 1.0265x over previous
import jax
import jax.numpy as jnp
from jax.experimental import pallas as pl
from jax.experimental.pallas import tpu as pltpu


def _body(step_ref, pat_ref, x_ref, tpe_ref, spe_ref, qpe_ref, o_ref):
    b = pl.program_id(1)
    s = step_ref[0]
    idx = pat_ref[b] % 64
    td = tpe_ref.shape[1]
    sd = spe_ref.shape[2]
    t_row = tpe_ref[s, :]
    q_row = qpe_ref[idx, :]
    o_ref[..., :td] = x_ref[..., :td] + t_row[None, None, None, :]
    o_ref[..., td:td + sd] = x_ref[..., td:td + sd] + spe_ref[...][None]
    o_ref[..., td + sd:] = x_ref[..., td + sd:] + q_row[None, None, None, :]


def kernel(x, temporal_step, sequence_pattern, temporal_pe, spatial_pe, sequence_pe):
    B, H, W, D = x.shape
    TD = temporal_pe.shape[1]
    SD = spatial_pe.shape[2]
    QD = sequence_pe.shape[1]
    RB = 32
    R = H // RB

    step = jnp.asarray(temporal_step, jnp.int32).reshape(1)
    pat = jnp.asarray(sequence_pattern, jnp.int32)

    grid_spec = pltpu.PrefetchScalarGridSpec(
        num_scalar_prefetch=2,
        grid=(R, B),
        in_specs=[
            pl.BlockSpec((1, RB, W, D), lambda r, b, *_: (b, r, 0, 0)),
            pl.BlockSpec(temporal_pe.shape, lambda r, b, *_: (0, 0)),
            pl.BlockSpec((RB, W, SD), lambda r, b, *_: (r, 0, 0)),
            pl.BlockSpec(sequence_pe.shape, lambda r, b, *_: (0, 0)),
        ],
        out_specs=pl.BlockSpec((1, RB, W, D), lambda r, b, *_: (b, r, 0, 0)),
    )
    return pl.pallas_call(
        _body,
        grid_spec=grid_spec,
        out_shape=jax.ShapeDtypeStruct(x.shape, x.dtype),
    )(step, pat, x, temporal_pe, spatial_pe, sequence_pe)


# speedup vs baseline: 1.6914x; 1.0262x over previous
import jax
import jax.numpy as jnp
from jax.experimental import pallas as pl
from jax.experimental.pallas import tpu as pltpu


def _body(step_ref, pat_ref, x_ref, tpe_ref, spe_ref, qpe_ref, o_ref):
    b = pl.program_id(1)
    s = step_ref[0]
    idx = pat_ref[b] % 64
    td = tpe_ref.shape[1]
    sd = spe_ref.shape[2]
    t_row = tpe_ref[s, :]
    q_row = qpe_ref[idx, :]
    o_ref[..., :td] = x_ref[..., :td] + t_row[None, None, None, :]
    o_ref[..., td:td + sd] = x_ref[..., td:td + sd] + spe_ref[...][None]
    o_ref[..., td + sd:] = x_ref[..., td + sd:] + q_row[None, None, None, :]


def kernel(x, temporal_step, sequence_pattern, temporal_pe, spatial_pe, sequence_pe):
    B, H, W, D = x.shape
    TD = temporal_pe.shape[1]
    SD = spatial_pe.shape[2]
    QD = sequence_pe.shape[1]
    RB = 64
    R = H // RB

    step = jnp.asarray(temporal_step, jnp.int32).reshape(1)
    pat = jnp.asarray(sequence_pattern, jnp.int32)

    grid_spec = pltpu.PrefetchScalarGridSpec(
        num_scalar_prefetch=2,
        grid=(R, B),
        in_specs=[
            pl.BlockSpec((1, RB, W, D), lambda r, b, *_: (b, r, 0, 0)),
            pl.BlockSpec(temporal_pe.shape, lambda r, b, *_: (0, 0)),
            pl.BlockSpec((RB, W, SD), lambda r, b, *_: (r, 0, 0)),
            pl.BlockSpec(sequence_pe.shape, lambda r, b, *_: (0, 0)),
        ],
        out_specs=pl.BlockSpec((1, RB, W, D), lambda r, b, *_: (b, r, 0, 0)),
    )
    return pl.pallas_call(
        _body,
        grid_spec=grid_spec,
        out_shape=jax.ShapeDtypeStruct(x.shape, x.dtype),
    )(step, pat, x, temporal_pe, spatial_pe, sequence_pe)
